# baseline (speedup 1.0000x reference)
import functools

import jax
import jax.numpy as jnp
from jax import lax
from jax.experimental import pallas as pl
from jax.experimental.pallas import tpu as pltpu
from jax.experimental.pallas import tpu_sc as plsc

N = 10000
D = 128
E = 320000

NC = 2
NS = 16
NW = NC * NS

NPAD = 10112
HALF = 5120
TRASH = 10000
CHD = 128
KCHD = 80
CH = 32
NBUF = 2
KCH2 = 640
SCH = 8
NSTG = KCH2 // SCH
EPAD = NW * KCHD * CHD
RPT = NPAD // NS
HRPT = HALF // NS
NPADD = 10240
RPTD = NPADD // NS
EDG_W = EPAD // NW
SBLK = SCH * CH
CAP = EDG_W + 2 * SBLK
LTOT = NW * 4 * CAP


def _mesh():
    return plsc.VectorSubcoreMesh(core_axis_name="c", subcore_axis_name="s")


def _deg_body(dst_hbm, zeros_hbm, out_hbm, didx_v, ones_v, deg_sh, ssem):
    c = lax.axis_index("c")
    s = lax.axis_index("s")
    wid = c * NS + s
    for i in range(CHD // 16):
        ones_v[pl.ds(i * 16, 16)] = jnp.full((16,), 1.0, jnp.float32)
    pltpu.sync_copy(zeros_hbm.at[pl.ds(s * RPTD, RPTD)],
                    deg_sh.at[pl.ds(s * RPTD, RPTD)])
    pltpu.sync_copy(dst_hbm.at[wid], didx_v)
    plsc.subcore_barrier()

    def fire(j, carry):
        pltpu.async_copy(ones_v, deg_sh.at[didx_v.at[j]], ssem, add=True)
        return carry

    lax.fori_loop(0, KCHD, fire, 0)

    def drain(j, carry):
        pltpu.make_async_copy(ones_v, deg_sh.at[didx_v.at[0]], ssem).wait()
        return carry

    lax.fori_loop(0, KCHD, drain, 0)
    plsc.subcore_barrier()
    pltpu.sync_copy(deg_sh.at[pl.ds(s * RPTD, RPTD)],
                    out_hbm.at[pl.ds(c * NPADD + s * RPTD, RPTD)])


def _deg_counts(dst3, zeros1):
    k = pl.kernel(
        _deg_body,
        mesh=_mesh(),
        out_type=jax.ShapeDtypeStruct((NC * NPADD,), jnp.float32),
        scratch_types=[
            pltpu.VMEM((KCHD, CHD), jnp.int32),
            pltpu.VMEM((CHD,), jnp.float32),
            pltpu.VMEM_SHARED((NPADD,), jnp.float32),
            pltpu.SemaphoreType.DMA,
        ],
    )
    return k(dst3, zeros1)


def _agg_body(hs_hbm, src_hbm, dst_hbm, zeros_hbm, out_hbm,
              sst_v, dst_v, sl0, sl1, dl0, dl1,
              rows0, rows1, hs_sh, acc_sh, gsem, ssem):
    c = lax.axis_index("c")
    s = lax.axis_index("s")
    base = c * HALF
    slocs = [sl0, sl1]
    dlocs = [dl0, dl1]
    bufs = [rows0, rows1]
    pltpu.sync_copy(zeros_hbm.at[pl.ds(s * RPT, RPT)],
                    acc_sh.at[pl.ds(s * RPT, RPT)])
    pltpu.sync_copy(hs_hbm.at[pl.ds(base + s * HRPT, HRPT)],
                    hs_sh.at[pl.ds(s * HRPT, HRPT)])
    plsc.subcore_barrier()

    def prep(k):
        for q in range(CH // 16):
            sv = sst_v[pl.ds(k * CH + q * 16, 16)]
            dv = dst_v[pl.ds(k * CH + q * 16, 16)]
            valid = (sv >= base) & (sv < base + HALF)
            slocs[k % 2][pl.ds(q * 16, 16)] = jnp.where(valid, sv - base, 0)
            dlocs[k % 2][pl.ds(q * 16, 16)] = jnp.where(valid, dv, TRASH)

    def stage(h, carry):
        pltpu.sync_copy(src_hbm.at[s, h], sst_v)
        pltpu.sync_copy(dst_hbm.at[s, h], dst_v)
        prep(0)
        pltpu.async_copy(hs_sh.at[slocs[0]], bufs[0], gsem)
        for k in range(SCH):
            b = k % 2
            pltpu.make_async_copy(hs_sh.at[slocs[b]], bufs[b], gsem).wait()
            pltpu.async_copy(bufs[b], acc_sh.at[dlocs[b]], ssem, add=True)
            if k >= 1:
                pltpu.make_async_copy(bufs[1 - b], acc_sh.at[dlocs[1 - b]],
                                      ssem).wait()
            if k + 1 < SCH:
                prep(k + 1)
                pltpu.async_copy(hs_sh.at[slocs[1 - b]], bufs[1 - b], gsem)
        pltpu.make_async_copy(bufs[(SCH - 1) % 2],
                              acc_sh.at[dlocs[(SCH - 1) % 2]], ssem).wait()
        return carry

    lax.fori_loop(0, NSTG, stage, 0)
    plsc.subcore_barrier()
    pltpu.sync_copy(acc_sh.at[pl.ds(s * RPT, RPT)],
                    out_hbm.at[c, pl.ds(s * RPT, RPT)])


def _edge_aggregate(hs_split, src3, dst3, zeros2):
    k = pl.kernel(
        _agg_body,
        mesh=_mesh(),
        out_type=jax.ShapeDtypeStruct((NC, NPAD, D), jnp.float32),
        scratch_types=[
            pltpu.VMEM((SBLK,), jnp.int32),
            pltpu.VMEM((SBLK,), jnp.int32),
            pltpu.VMEM((CH,), jnp.int32),
            pltpu.VMEM((CH,), jnp.int32),
            pltpu.VMEM((CH,), jnp.int32),
            pltpu.VMEM((CH,), jnp.int32),
            pltpu.VMEM((CH, D), jnp.float32),
            pltpu.VMEM((CH, D), jnp.float32),
            pltpu.VMEM_SHARED((HALF, D), jnp.float32),
            pltpu.VMEM_SHARED((NPAD, D), jnp.float32),
            pltpu.SemaphoreType.DMA,
            pltpu.SemaphoreType.DMA,
        ],
    )
    return k(hs_split, src3, dst3, zeros2)


_RB = 1264
_GRID = NPAD // _RB


def _k_dinv_hs(degp_ref, x_ref, w_ref, dinv_ref, hs_ref):
    deg = 1.0 + degp_ref[0] + degp_ref[1]
    dinv = lax.rsqrt(deg)
    dinv_ref[...] = dinv
    hs_ref[...] = jnp.dot(x_ref[...], w_ref[...],
                          preferred_element_type=jnp.float32) * dinv


def _dinv_hs(degp, x, w1):
    return pl.pallas_call(
        _k_dinv_hs,
        grid=(_GRID,),
        in_specs=[
            pl.BlockSpec((NC, _RB, 1), lambda i: (0, i, 0)),
            pl.BlockSpec((_RB, D), lambda i: (i, 0)),
            pl.BlockSpec((D, D), lambda i: (0, 0)),
        ],
        out_specs=[
            pl.BlockSpec((_RB, 1), lambda i: (i, 0)),
            pl.BlockSpec((_RB, D), lambda i: (i, 0)),
        ],
        out_shape=[
            jax.ShapeDtypeStruct((NPAD, 1), jnp.float32),
            jax.ShapeDtypeStruct((NPAD, D), jnp.float32),
        ],
    )(degp, x, w1)


def _k_layer_mid(p_ref, hs_ref, dinv_ref, b_ref, w_ref, o_ref):
    agg = (p_ref[0] + p_ref[1] + hs_ref[...]) * dinv_ref[...] + b_ref[...]
    h1 = jnp.maximum(agg, 0.0)
    o_ref[...] = jnp.dot(h1, w_ref[...],
                         preferred_element_type=jnp.float32) * dinv_ref[...]


def _layer_mid(p1, hs1, dinv, b1, w2):
    return pl.pallas_call(
        _k_layer_mid,
        grid=(_GRID,),
        in_specs=[
            pl.BlockSpec((NC, _RB, D), lambda i: (0, i, 0)),
            pl.BlockSpec((_RB, D), lambda i: (i, 0)),
            pl.BlockSpec((_RB, 1), lambda i: (i, 0)),
            pl.BlockSpec((1, D), lambda i: (0, 0)),
            pl.BlockSpec((D, D), lambda i: (0, 0)),
        ],
        out_specs=pl.BlockSpec((_RB, D), lambda i: (i, 0)),
        out_shape=jax.ShapeDtypeStruct((NPAD, D), jnp.float32),
    )(p1, hs1, dinv, b1, w2)


def _k_layer_out(p_ref, hs_ref, dinv_ref, b_ref, o_ref):
    o_ref[...] = (p_ref[0] + p_ref[1] + hs_ref[...]) * dinv_ref[...] + b_ref[...]


def _layer_out(p2, hs2, dinv, b2):
    return pl.pallas_call(
        _k_layer_out,
        grid=(_GRID,),
        in_specs=[
            pl.BlockSpec((NC, _RB, D), lambda i: (0, i, 0)),
            pl.BlockSpec((_RB, D), lambda i: (i, 0)),
            pl.BlockSpec((_RB, 1), lambda i: (i, 0)),
            pl.BlockSpec((1, D), lambda i: (0, 0)),
        ],
        out_specs=pl.BlockSpec((_RB, D), lambda i: (i, 0)),
        out_shape=jax.ShapeDtypeStruct((NPAD, D), jnp.float32),
    )(p2, hs2, dinv, b2)


def _split_src_halves(hs):
    return jnp.concatenate(
        [hs, jnp.zeros((2 * HALF - NPAD, D), hs.dtype)])


def kernel(x, edge_index, lgraph, W1, b1, W2, b2):
    src = edge_index[0]
    dst = edge_index[1]
    pad = EPAD - E
    srcp = jnp.concatenate([src, jnp.zeros((pad,), jnp.int32)])
    dstp = jnp.concatenate([dst, jnp.full((pad,), TRASH, jnp.int32)])
    src3 = srcp.reshape(NS, NSTG, SBLK)
    dst3d = dstp.reshape(NW, KCHD, CHD)
    dst3a = dstp.reshape(NS, NSTG, SBLK)
    xp = jnp.zeros((NPAD, D), x.dtype).at[:N].set(x)
    zeros2 = jnp.zeros((NPAD, D), jnp.float32)
    zeros1 = jnp.zeros((NPADD,), jnp.float32)

    degp = _deg_counts(dst3d, zeros1).reshape(NC, NPADD)[:, :NPAD]
    dinv, hs1 = _dinv_hs(degp.reshape(NC, NPAD, 1), xp, W1)
    p1 = _edge_aggregate(_split_src_halves(hs1), src3, dst3a, zeros2)
    hs2 = _layer_mid(p1, hs1, dinv, b1.reshape(1, D), W2)
    p2 = _edge_aggregate(_split_src_halves(hs2), src3, dst3a, zeros2)
    zp = _layer_out(p2, hs2, dinv, b2.reshape(1, D))
    return (zp[:N], edge_index)


# speedup vs baseline: 1.0566x; 1.0566x over previous
import functools

import jax
import jax.numpy as jnp
from jax import lax
from jax.experimental import pallas as pl
from jax.experimental.pallas import tpu as pltpu
from jax.experimental.pallas import tpu_sc as plsc

N = 10000
D = 128
E = 320000

NC = 2
NS = 16
NW = NC * NS

NPAD = 10112
HALF = 5120
TRASH = 10000
CHD = 128
KCHD = 80
CH = 32
NBUF = 2
KCH2 = 640
SCH = 8
NSTG = KCH2 // SCH
EPAD = NW * KCHD * CHD
RPT = NPAD // NS
HRPT = HALF // NS
NPADD = 10240
RPTD = NPADD // NS
EDG_W = EPAD // NW
SBLK = SCH * CH
CAP = EDG_W + 2 * SBLK
LTOT = NW * 4 * CAP


def _mesh():
    return plsc.VectorSubcoreMesh(core_axis_name="c", subcore_axis_name="s")


def _deg_body(dst_hbm, zeros_hbm, out_hbm, didx_v, ones_v, deg_sh, ssem):
    c = lax.axis_index("c")
    s = lax.axis_index("s")
    wid = c * NS + s
    for i in range(CHD // 16):
        ones_v[pl.ds(i * 16, 16)] = jnp.full((16,), 1.0, jnp.float32)
    pltpu.sync_copy(zeros_hbm.at[pl.ds(s * RPTD, RPTD)],
                    deg_sh.at[pl.ds(s * RPTD, RPTD)])
    pltpu.sync_copy(dst_hbm.at[wid], didx_v)
    plsc.subcore_barrier()

    def fire(j, carry):
        pltpu.async_copy(ones_v, deg_sh.at[didx_v.at[j]], ssem, add=True)
        return carry

    lax.fori_loop(0, KCHD, fire, 0)

    def drain(j, carry):
        pltpu.make_async_copy(ones_v, deg_sh.at[didx_v.at[0]], ssem).wait()
        return carry

    lax.fori_loop(0, KCHD, drain, 0)
    plsc.subcore_barrier()
    pltpu.sync_copy(deg_sh.at[pl.ds(s * RPTD, RPTD)],
                    out_hbm.at[pl.ds(c * NPADD + s * RPTD, RPTD)])


def _deg_counts(dst3, zeros1):
    k = pl.kernel(
        _deg_body,
        mesh=_mesh(),
        out_type=jax.ShapeDtypeStruct((NC * NPADD,), jnp.float32),
        scratch_types=[
            pltpu.VMEM((KCHD, CHD), jnp.int32),
            pltpu.VMEM((CHD,), jnp.float32),
            pltpu.VMEM_SHARED((NPADD,), jnp.float32),
            pltpu.SemaphoreType.DMA,
        ],
    )
    return k(dst3, zeros1)


def _agg_body(hs_hbm, idx_hbm, zeros_hbm, out_hbm,
              comb_v, sl0, sl1, dl0, dl1,
              rows0, rows1, hs_sh, acc_sh, gsem, ssem):
    c = lax.axis_index("c")
    s = lax.axis_index("s")
    base = c * HALF
    slocs = [sl0, sl1]
    dlocs = [dl0, dl1]
    bufs = [rows0, rows1]
    pltpu.sync_copy(zeros_hbm.at[pl.ds(s * RPT, RPT)],
                    acc_sh.at[pl.ds(s * RPT, RPT)])
    pltpu.sync_copy(hs_hbm.at[pl.ds(base + s * HRPT, HRPT)],
                    hs_sh.at[pl.ds(s * HRPT, HRPT)])
    plsc.subcore_barrier()

    def prep(k):
        for q in range(CH // 16):
            sv = comb_v[pl.ds(k * CH + q * 16, 16)]
            dv = comb_v[pl.ds(SBLK + k * CH + q * 16, 16)]
            valid = (sv >= base) & (sv < base + HALF)
            slocs[k % 2][pl.ds(q * 16, 16)] = jnp.where(valid, sv - base, 0)
            dlocs[k % 2][pl.ds(q * 16, 16)] = jnp.where(valid, dv, TRASH)

    def stage(h, carry):
        pltpu.sync_copy(idx_hbm.at[s, h], comb_v)
        prep(0)
        pltpu.async_copy(hs_sh.at[slocs[0]], bufs[0], gsem)
        for k in range(SCH):
            b = k % 2
            pltpu.make_async_copy(hs_sh.at[slocs[b]], bufs[b], gsem).wait()
            pltpu.async_copy(bufs[b], acc_sh.at[dlocs[b]], ssem, add=True)
            if k >= 1:
                pltpu.make_async_copy(bufs[1 - b], acc_sh.at[dlocs[1 - b]],
                                      ssem).wait()
            if k + 1 < SCH:
                prep(k + 1)
                pltpu.async_copy(hs_sh.at[slocs[1 - b]], bufs[1 - b], gsem)
        pltpu.make_async_copy(bufs[(SCH - 1) % 2],
                              acc_sh.at[dlocs[(SCH - 1) % 2]], ssem).wait()
        return carry

    lax.fori_loop(0, NSTG, stage, 0)
    plsc.subcore_barrier()
    pltpu.sync_copy(acc_sh.at[pl.ds(s * RPT, RPT)],
                    out_hbm.at[c, pl.ds(s * RPT, RPT)])


def _edge_aggregate(hs_split, idx3, zeros2):
    k = pl.kernel(
        _agg_body,
        mesh=_mesh(),
        out_type=jax.ShapeDtypeStruct((NC, NPAD, D), jnp.float32),
        scratch_types=[
            pltpu.VMEM((2 * SBLK,), jnp.int32),
            pltpu.VMEM((CH,), jnp.int32),
            pltpu.VMEM((CH,), jnp.int32),
            pltpu.VMEM((CH,), jnp.int32),
            pltpu.VMEM((CH,), jnp.int32),
            pltpu.VMEM((CH, D), jnp.float32),
            pltpu.VMEM((CH, D), jnp.float32),
            pltpu.VMEM_SHARED((HALF, D), jnp.float32),
            pltpu.VMEM_SHARED((NPAD, D), jnp.float32),
            pltpu.SemaphoreType.DMA,
            pltpu.SemaphoreType.DMA,
        ],
    )
    return k(hs_split, idx3, zeros2)


_RB = 1264
_GRID = NPAD // _RB


def _k_dinv_hs(degp_ref, x_ref, w_ref, dinv_ref, hs_ref):
    deg = 1.0 + degp_ref[0] + degp_ref[1]
    dinv = lax.rsqrt(deg)
    dinv_ref[...] = dinv
    hs_ref[...] = jnp.dot(x_ref[...], w_ref[...],
                          preferred_element_type=jnp.float32) * dinv


def _dinv_hs(degp, x, w1):
    return pl.pallas_call(
        _k_dinv_hs,
        grid=(_GRID,),
        in_specs=[
            pl.BlockSpec((NC, _RB, 1), lambda i: (0, i, 0)),
            pl.BlockSpec((_RB, D), lambda i: (i, 0)),
            pl.BlockSpec((D, D), lambda i: (0, 0)),
        ],
        out_specs=[
            pl.BlockSpec((_RB, 1), lambda i: (i, 0)),
            pl.BlockSpec((_RB, D), lambda i: (i, 0)),
        ],
        out_shape=[
            jax.ShapeDtypeStruct((NPAD, 1), jnp.float32),
            jax.ShapeDtypeStruct((NPAD, D), jnp.float32),
        ],
    )(degp, x, w1)


def _k_layer_mid(p_ref, hs_ref, dinv_ref, b_ref, w_ref, o_ref):
    agg = (p_ref[0] + p_ref[1] + hs_ref[...]) * dinv_ref[...] + b_ref[...]
    h1 = jnp.maximum(agg, 0.0)
    o_ref[...] = jnp.dot(h1, w_ref[...],
                         preferred_element_type=jnp.float32) * dinv_ref[...]


def _layer_mid(p1, hs1, dinv, b1, w2):
    return pl.pallas_call(
        _k_layer_mid,
        grid=(_GRID,),
        in_specs=[
            pl.BlockSpec((NC, _RB, D), lambda i: (0, i, 0)),
            pl.BlockSpec((_RB, D), lambda i: (i, 0)),
            pl.BlockSpec((_RB, 1), lambda i: (i, 0)),
            pl.BlockSpec((1, D), lambda i: (0, 0)),
            pl.BlockSpec((D, D), lambda i: (0, 0)),
        ],
        out_specs=pl.BlockSpec((_RB, D), lambda i: (i, 0)),
        out_shape=jax.ShapeDtypeStruct((NPAD, D), jnp.float32),
    )(p1, hs1, dinv, b1, w2)


def _k_layer_out(p_ref, hs_ref, dinv_ref, b_ref, o_ref):
    o_ref[...] = (p_ref[0] + p_ref[1] + hs_ref[...]) * dinv_ref[...] + b_ref[...]


def _layer_out(p2, hs2, dinv, b2):
    return pl.pallas_call(
        _k_layer_out,
        grid=(_GRID,),
        in_specs=[
            pl.BlockSpec((NC, _RB, D), lambda i: (0, i, 0)),
            pl.BlockSpec((_RB, D), lambda i: (i, 0)),
            pl.BlockSpec((_RB, 1), lambda i: (i, 0)),
            pl.BlockSpec((1, D), lambda i: (0, 0)),
        ],
        out_specs=pl.BlockSpec((_RB, D), lambda i: (i, 0)),
        out_shape=jax.ShapeDtypeStruct((NPAD, D), jnp.float32),
    )(p2, hs2, dinv, b2)


def _split_src_halves(hs):
    return jnp.concatenate(
        [hs, jnp.zeros((2 * HALF - NPAD, D), hs.dtype)])


def kernel(x, edge_index, lgraph, W1, b1, W2, b2):
    src = edge_index[0]
    dst = edge_index[1]
    pad = EPAD - E
    srcp = jnp.concatenate([src, jnp.zeros((pad,), jnp.int32)])
    dstp = jnp.concatenate([dst, jnp.full((pad,), TRASH, jnp.int32)])
    dst3d = dstp.reshape(NW, KCHD, CHD)
    idx3 = jnp.stack([srcp.reshape(NS, NSTG, SBLK),
                      dstp.reshape(NS, NSTG, SBLK)],
                     axis=2).reshape(NS, NSTG, 2 * SBLK)
    xp = jnp.zeros((NPAD, D), x.dtype).at[:N].set(x)
    zeros2 = jnp.zeros((NPAD, D), jnp.float32)
    zeros1 = jnp.zeros((NPADD,), jnp.float32)

    degp = _deg_counts(dst3d, zeros1).reshape(NC, NPADD)[:, :NPAD]
    dinv, hs1 = _dinv_hs(degp.reshape(NC, NPAD, 1), xp, W1)
    p1 = _edge_aggregate(_split_src_halves(hs1), idx3, zeros2)
    hs2 = _layer_mid(p1, hs1, dinv, b1.reshape(1, D), W2)
    p2 = _edge_aggregate(_split_src_halves(hs2), idx3, zeros2)
    zp = _layer_out(p2, hs2, dinv, b2.reshape(1, D))
    return (zp[:N], edge_index)
